# baseline (speedup 1.0000x reference)
import functools

import jax
import jax.numpy as jnp
from jax import lax
from jax.experimental import pallas as pl
from jax.experimental.pallas import tpu as pltpu
from jax.experimental.pallas import tpu_sc as plsc

_N = 10000
_D = 128
_H = 256
_BN_EPS = 1e-5

_W = 128
_NC = 2
_NS = 16
_NWORK = _NC * _NS
_ACC_ROWS = 10240
_ZROWS = _ACC_ROWS // _NS


def _sc_aggregate(x, edges, zeros, nwin):
    niter = -(-nwin // _NWORK)
    niter = -(-niter // 4) * 4
    mesh = plsc.VectorSubcoreMesh(core_axis_name="c", subcore_axis_name="s")

    @functools.partial(
        pl.kernel,
        out_type=jax.ShapeDtypeStruct((_NC, _N, _D), jnp.float32),
        mesh=mesh,
        scratch_types=(
            [pltpu.VMEM((2, _W), jnp.int32)] * 4
            + [pltpu.VMEM((_W, _D), jnp.float32)] * 2
            + [pltpu.SemaphoreType.DMA] * 8
            + [pltpu.VMEM_SHARED((_ACC_ROWS, _D), jnp.float32)]
        ),
    )
    def agg_kernel(x_hbm, e_hbm, z_hbm, out_hbm,
                   ib0, ib1, ib2, ib3, rows0, rows1,
                   is0, is1, is2, is3, gs0, gs1, ss0, ss1, acc):
        cid = lax.axis_index("c")
        sid = lax.axis_index("s")
        wid = cid * _NS + sid
        ibuf = (ib0, ib1, ib2, ib3)
        isem = (is0, is1, is2, is3)
        rows = (rows0, rows1)
        gsem = (gs0, gs1)
        ssem = (ss0, ss1)

        def win(i):
            return (wid + i * _NWORK) * _W

        def idx_start(i, slot):
            pltpu.async_copy(e_hbm.at[:, pl.ds(win(i), _W)], ibuf[slot],
                             isem[slot])

        def idx_wait(i, slot):
            pltpu.make_async_copy(e_hbm.at[:, pl.ds(win(i), _W)], ibuf[slot],
                                  isem[slot]).wait()

        def gather_start(slot):
            pltpu.async_copy(x_hbm.at[ibuf[slot % 4].at[0]], rows[slot % 2],
                             gsem[slot % 2])

        def gather_wait(slot):
            pltpu.make_async_copy(x_hbm.at[ibuf[slot % 4].at[0]],
                                  rows[slot % 2], gsem[slot % 2]).wait()

        pltpu.sync_copy(z_hbm, acc.at[pl.ds(sid * _ZROWS, _ZROWS)])
        idx_start(0, 0)
        idx_start(1, 1)
        idx_wait(0, 0)
        gather_start(0)
        plsc.subcore_barrier()

        nvalid = (nwin - wid + _NWORK - 1) // _NWORK

        @pl.loop(0, niter, step=4)
        def _(base):
            for k in range(4):
                i = base + k
                @pl.when(i + 2 < nvalid)
                def _():
                    idx_start(i + 2, (k + 2) % 4)

                @pl.when(i + 1 < nvalid)
                def _():
                    idx_wait(i + 1, (k + 1) % 4)

                    @pl.when(i >= 1)
                    def _():
                        pltpu.make_async_copy(
                            rows[(k + 1) % 2],
                            acc.at[ibuf[(k + 3) % 4].at[1]],
                            ssem[(k + 1) % 2]).wait()

                    gather_start(k + 1)

                @pl.when(i < nvalid)
                def _():
                    gather_wait(k)
                    pltpu.async_copy(rows[k % 2], acc.at[ibuf[k % 4].at[1]],
                                     ssem[k % 2], add=True)

        pltpu.make_async_copy(rows[0], acc.at[ibuf[0].at[1]], ssem[0]).wait()
        pltpu.make_async_copy(rows[1], acc.at[ibuf[1].at[1]], ssem[1]).wait()

        plsc.subcore_barrier()
        rpw = 624
        pltpu.sync_copy(acc.at[pl.ds(sid * rpw, rpw)],
                        out_hbm.at[cid].at[pl.ds(sid * rpw, rpw)])

        @pl.when(sid == _NS - 1)
        def _():
            tail = _NS * rpw
            pltpu.sync_copy(acc.at[pl.ds(tail, _N - tail)],
                            out_hbm.at[cid].at[pl.ds(tail, _N - tail)])

    return agg_kernel(x, edges, zeros)


_NB = 10
_BR = _N // _NB


def _col_sums(tb):
    ones = jnp.ones((1, tb.shape[0]), jnp.bfloat16)
    s1 = jnp.dot(ones, tb, preferred_element_type=jnp.float32)
    s2 = jnp.dot(ones, tb * tb, preferred_element_type=jnp.float32)
    return s1, s2


def _bn_ab(s1, s2, g, be):
    inv_n = 1.0 / _N
    mu = s1 * inv_n
    var = s2 * inv_n - mu * mu
    a = g * lax.rsqrt(var + _BN_EPS)
    return a, be - mu * a


def _mlp_body(eps_ref, x_ref, agg_ref, w1_ref, g1_ref, be1_ref,
              w2_ref, g2_ref, be2_ref, o_ref,
              t_s, u_s, s1_s, s2_s, a1_s, c1_s, a2_s, c2_s):
    p = pl.program_id(0)
    j = pl.program_id(1)
    rows = pl.ds(j * _BR, _BR)

    @pl.when(p == 0)
    def _():
        h = (1.0 + eps_ref[0]) * x_ref[...] + agg_ref[0] + agg_ref[1]
        t = jnp.dot(h.astype(jnp.bfloat16), w1_ref[...].astype(jnp.bfloat16),
                    preferred_element_type=jnp.float32)
        t_s[rows, :] = t
        s1, s2 = _col_sums(t.astype(jnp.bfloat16))

        @pl.when(j == 0)
        def _():
            s1_s[:, :_H] = s1
            s2_s[:, :_H] = s2

        @pl.when(j > 0)
        def _():
            s1_s[:, :_H] += s1
            s2_s[:, :_H] += s2

    @pl.when(p == 1)
    def _():
        @pl.when(j == 0)
        def _():
            a1, c1 = _bn_ab(s1_s[:, :_H], s2_s[:, :_H],
                            g1_ref[...], be1_ref[...])
            a1_s[...] = a1
            c1_s[...] = c1

        t = jnp.maximum(t_s[rows, :] * a1_s[...] + c1_s[...], 0.0)
        u = jnp.dot(t.astype(jnp.bfloat16), w2_ref[...].astype(jnp.bfloat16),
                    preferred_element_type=jnp.float32)
        u_s[rows, :] = u
        r1, r2 = _col_sums(u.astype(jnp.bfloat16))

        @pl.when(j == 0)
        def _():
            s1_s[:, _H:] = r1
            s2_s[:, _H:] = r2

        @pl.when(j > 0)
        def _():
            s1_s[:, _H:] += r1
            s2_s[:, _H:] += r2

    @pl.when(p == 2)
    def _():
        @pl.when(j == 0)
        def _():
            a2, c2 = _bn_ab(s1_s[:, _H:], s2_s[:, _H:],
                            g2_ref[...], be2_ref[...])
            a2_s[...] = a2
            c2_s[...] = c2

        o_ref[...] = jnp.maximum(u_s[rows, :] * a2_s[...] + c2_s[...], 0.0)


def _mlp(eps, x, aggpair, W1, g1, be1, W2, g2, be2):
    def first_phase(p, j):
        return jnp.where(p == 0, j, 0)

    return pl.pallas_call(
        _mlp_body,
        grid=(3, _NB),
        out_shape=jax.ShapeDtypeStruct((_N, _D), jnp.float32),
        in_specs=[
            pl.BlockSpec(memory_space=pltpu.SMEM),
            pl.BlockSpec((_BR, _D), lambda p, j: (first_phase(p, j), 0)),
            pl.BlockSpec((_NC, _BR, _D),
                         lambda p, j: (0, first_phase(p, j), 0)),
            pl.BlockSpec((_D, _H), lambda p, j: (0, 0)),
            pl.BlockSpec((1, _H), lambda p, j: (0, 0)),
            pl.BlockSpec((1, _H), lambda p, j: (0, 0)),
            pl.BlockSpec((_H, _D), lambda p, j: (0, 0)),
            pl.BlockSpec((1, _D), lambda p, j: (0, 0)),
            pl.BlockSpec((1, _D), lambda p, j: (0, 0)),
        ],
        out_specs=pl.BlockSpec((_BR, _D),
                               lambda p, j: (jnp.where(p == 2, j, 0), 0)),
        scratch_shapes=[
            pltpu.VMEM((_N, _H), jnp.float32),
            pltpu.VMEM((_N, _D), jnp.float32),
            pltpu.VMEM((1, _H + _D), jnp.float32),
            pltpu.VMEM((1, _H + _D), jnp.float32),
            pltpu.VMEM((1, _H), jnp.float32),
            pltpu.VMEM((1, _H), jnp.float32),
            pltpu.VMEM((1, _D), jnp.float32),
            pltpu.VMEM((1, _D), jnp.float32),
        ],
    )(eps, x, aggpair, W1, g1, be1, W2, g2, be2)


def kernel(x, edge_index, eps, W1, b1, g1, be1, W2, b2, g2, be2):
    E = edge_index.shape[1]
    rem = E % _W
    edges = edge_index
    if rem:
        pad = _W - rem
        ar = jnp.arange(pad, dtype=jnp.int32)
        edges = jnp.concatenate(
            [edge_index,
             jnp.stack([ar % _N, _N + ar % (_ACC_ROWS - _N)])], axis=1)
    nwin = (E + _W - 1) // _W
    zeros = jnp.zeros((_ZROWS, _D), jnp.float32)
    aggpair = _sc_aggregate(x, edges, zeros, nwin)
    return _mlp(jnp.reshape(eps, (1,)), x, aggpair,
                W1, jnp.reshape(g1, (1, _H)), jnp.reshape(be1, (1, _H)),
                W2, jnp.reshape(g2, (1, _D)), jnp.reshape(be2, (1, _D)))


# speedup vs baseline: 1.1005x; 1.1005x over previous
import functools

import jax
import jax.numpy as jnp
from jax import lax
from jax.experimental import pallas as pl
from jax.experimental.pallas import tpu as pltpu
from jax.experimental.pallas import tpu_sc as plsc

_N = 10000
_D = 128
_H = 256
_BN_EPS = 1e-5

_W = 128
_NC = 2
_NS = 16
_NWORK = _NC * _NS
_ACC_ROWS = 10096
_ZROWS = _ACC_ROWS // _NS


def _sc_aggregate(x, edges, zeros, nwin):
    niter = -(-nwin // _NWORK)
    niter = -(-niter // 12) * 12
    mesh = plsc.VectorSubcoreMesh(core_axis_name="c", subcore_axis_name="s")

    @functools.partial(
        pl.kernel,
        out_type=jax.ShapeDtypeStruct((_NC, _N, _D), jnp.float32),
        mesh=mesh,
        scratch_types=(
            [pltpu.VMEM((2, _W), jnp.int32)] * 4
            + [pltpu.VMEM((_W, _D), jnp.float32)] * 3
            + [pltpu.SemaphoreType.DMA] * 10
            + [pltpu.VMEM_SHARED((_ACC_ROWS, _D), jnp.float32)]
        ),
    )
    def agg_kernel(x_hbm, e_hbm, z_hbm, out_hbm,
                   ib0, ib1, ib2, ib3, rows0, rows1, rows2,
                   is0, is1, is2, is3, gs0, gs1, gs2, ss0, ss1, ss2, acc):
        cid = lax.axis_index("c")
        sid = lax.axis_index("s")
        wid = cid * _NS + sid
        ibuf = (ib0, ib1, ib2, ib3)
        isem = (is0, is1, is2, is3)
        rows = (rows0, rows1, rows2)
        gsem = (gs0, gs1, gs2)
        ssem = (ss0, ss1, ss2)

        def win(i):
            return (wid + i * _NWORK) * _W

        def idx_start(i, slot):
            pltpu.async_copy(e_hbm.at[:, pl.ds(win(i), _W)], ibuf[slot],
                             isem[slot])

        def idx_wait(i, slot):
            pltpu.make_async_copy(e_hbm.at[:, pl.ds(win(i), _W)], ibuf[slot],
                                  isem[slot]).wait()

        def gather_start(slot):
            pltpu.async_copy(x_hbm.at[ibuf[slot % 4].at[0]], rows[slot % 3],
                             gsem[slot % 3])

        def gather_wait(slot):
            pltpu.make_async_copy(x_hbm.at[ibuf[slot % 4].at[0]],
                                  rows[slot % 3], gsem[slot % 3]).wait()

        def scatter_wait(slot):
            pltpu.make_async_copy(rows[slot % 3],
                                  acc.at[ibuf[slot % 4].at[1]],
                                  ssem[slot % 3]).wait()

        pltpu.sync_copy(z_hbm, acc.at[pl.ds(sid * _ZROWS, _ZROWS)])
        idx_start(0, 0)
        idx_start(1, 1)
        idx_wait(0, 0)
        gather_start(0)
        plsc.subcore_barrier()

        nvalid = (nwin - wid + _NWORK - 1) // _NWORK

        @pl.loop(0, niter, step=12)
        def _(base):
            for k in range(12):
                i = base + k
                @pl.when(i + 1 < nvalid)
                def _():
                    @pl.when(i >= 2)
                    def _():
                        scatter_wait(k + 1)

                    idx_wait(i + 1, (k + 1) % 4)

                @pl.when(i + 2 < nvalid)
                def _():
                    idx_start(i + 2, (k + 2) % 4)

                @pl.when(i + 1 < nvalid)
                def _():
                    gather_start(k + 1)

                @pl.when(i < nvalid)
                def _():
                    gather_wait(k)
                    pltpu.async_copy(rows[k % 3], acc.at[ibuf[k % 4].at[1]],
                                     ssem[k % 3], add=True)

        scatter_wait(0)
        scatter_wait(1)
        scatter_wait(2)

        plsc.subcore_barrier()
        rpw = 624
        pltpu.sync_copy(acc.at[pl.ds(sid * rpw, rpw)],
                        out_hbm.at[cid].at[pl.ds(sid * rpw, rpw)])

        @pl.when(sid == _NS - 1)
        def _():
            tail = _NS * rpw
            pltpu.sync_copy(acc.at[pl.ds(tail, _N - tail)],
                            out_hbm.at[cid].at[pl.ds(tail, _N - tail)])

    return agg_kernel(x, edges, zeros)


def _bn_coeffs(tb, t2b, g, be):
    ones = jnp.ones((1, tb.shape[0]), jnp.bfloat16)
    s1 = jnp.dot(ones, tb, preferred_element_type=jnp.float32)
    s2 = jnp.dot(ones, t2b, preferred_element_type=jnp.float32)
    inv_n = 1.0 / tb.shape[0]
    mu = s1 * inv_n
    var = s2 * inv_n - mu * mu
    a = g * lax.rsqrt(var + _BN_EPS)
    return a, be - mu * a


def _mlp_body(eps_ref, x_ref, agg_ref, w1_ref, b1_ref, g1_ref, be1_ref,
              w2_ref, b2_ref, g2_ref, be2_ref, o_ref):
    h = (1.0 + eps_ref[0]) * x_ref[...] + agg_ref[0] + agg_ref[1]
    t = jnp.dot(h.astype(jnp.bfloat16), w1_ref[...].astype(jnp.bfloat16),
                preferred_element_type=jnp.float32)
    tb = t.astype(jnp.bfloat16)
    a1, c1 = _bn_coeffs(tb, tb * tb, g1_ref[...], be1_ref[...])
    t = jnp.maximum(t * a1 + c1, 0.0)
    u = jnp.dot(t.astype(jnp.bfloat16), w2_ref[...].astype(jnp.bfloat16),
                preferred_element_type=jnp.float32)
    ub = u.astype(jnp.bfloat16)
    a2, c2 = _bn_coeffs(ub, ub * ub, g2_ref[...], be2_ref[...])
    o_ref[...] = jnp.maximum(u * a2 + c2, 0.0)


def _mlp(eps, x, aggpair, W1, b1, g1, be1, W2, b2, g2, be2):
    return pl.pallas_call(
        _mlp_body,
        out_shape=jax.ShapeDtypeStruct((_N, _D), jnp.float32),
        in_specs=[pl.BlockSpec(memory_space=pltpu.SMEM)]
                 + [pl.BlockSpec(memory_space=pltpu.VMEM)] * 10,
        out_specs=pl.BlockSpec(memory_space=pltpu.VMEM),
    )(eps, x, aggpair, W1, b1, g1, be1, W2, b2, g2, be2)


def kernel(x, edge_index, eps, W1, b1, g1, be1, W2, b2, g2, be2):
    E = edge_index.shape[1]
    rem = E % _W
    edges = edge_index
    if rem:
        pad = _W - rem
        ar = jnp.arange(pad, dtype=jnp.int32)
        edges = jnp.concatenate(
            [edge_index,
             jnp.stack([ar % _N, _N + ar % (_ACC_ROWS - _N)])], axis=1)
    nwin = (E + _W - 1) // _W
    zeros = jnp.zeros((_ZROWS, _D), jnp.float32)
    aggpair = _sc_aggregate(x, edges, zeros, nwin)
    return _mlp(jnp.reshape(eps, (1,)), x, aggpair,
                W1, jnp.reshape(b1, (1, _H)), jnp.reshape(g1, (1, _H)),
                jnp.reshape(be1, (1, _H)),
                W2, jnp.reshape(b2, (1, _D)), jnp.reshape(g2, (1, _D)),
                jnp.reshape(be2, (1, _D)))
